# baseline (speedup 1.0000x reference)
import functools

import jax
import jax.numpy as jnp
from jax import lax
from jax.experimental import pallas as pl
from jax.experimental.pallas import tpu as pltpu
from jax.experimental.pallas import tpu_sc as plsc

_INFO = plsc.get_sparse_core_info()
_NC, _NS, _L = _INFO.num_cores, _INFO.num_subcores, _INFO.num_lanes
_NW = _NC * _NS

_CHUNK = 64
_UNROLL = 4


def _build_lookup(b, s, d, n_chunks, n_pos):
    b_per_w = n_chunks * _CHUNK
    mesh = plsc.VectorSubcoreMesh(core_axis_name="c", subcore_axis_name="s")

    @functools.partial(
        pl.kernel,
        mesh=mesh,
        out_type=jax.ShapeDtypeStruct((b, s, d), jnp.float32),
        scratch_types=[
            pltpu.VMEM((b_per_w,), jnp.int32),
            pltpu.VMEM((b_per_w,), jnp.int32),
            pltpu.VMEM((b_per_w, d), jnp.float32),
            pltpu.VMEM((b_per_w, d), jnp.float32),
            pltpu.VMEM_SHARED((n_pos, d), jnp.float32),
            pltpu.VMEM_SHARED((_NS * b_per_w, d), jnp.float32),
            pltpu.SemaphoreType.DMA,
            pltpu.SemaphoreType.DMA,
            pltpu.SemaphoreType.DMA,
        ]
        + [pltpu.SemaphoreType.DMA for _ in range(2 * n_chunks)],
    )
    def emb_kernel(tok_hbm, pos_hbm, tokt_hbm, post_hbm, out_hbm,
                   tidx, pidx, trows, prows, post_sh, out_sh,
                   isem, ssem, fsem, *sems):
        gsems, csems = sems[:n_chunks], sems[n_chunks:]
        sid = lax.axis_index("s")
        wid = sid * _NC + lax.axis_index("c")
        bb = wid // (s // b_per_w)
        s0 = (wid % (s // b_per_w)) * b_per_w
        i0 = pltpu.async_copy(tok_hbm.at[bb, pl.ds(s0, b_per_w)], tidx, isem)
        i1 = pltpu.async_copy(pos_hbm.at[bb, pl.ds(s0, b_per_w)], pidx, isem)
        p_per_t = n_pos // _NS
        stg = pltpu.async_copy(
            post_hbm.at[pl.ds(sid * p_per_t, p_per_t)],
            post_sh.at[pl.ds(sid * p_per_t, p_per_t)],
            ssem)
        i0.wait()
        i1.wait()
        tok_gathers = []
        for j in range(n_chunks):
            sl = pl.ds(j * _CHUNK, _CHUNK)
            tok_gathers.append(
                pltpu.async_copy(tokt_hbm.at[tidx.at[sl]], trows.at[sl], gsems[j]))
        stg.wait()
        plsc.subcore_barrier()
        pos_gathers = []
        for j in range(n_chunks):
            sl = pl.ds(j * _CHUNK, _CHUNK)
            pos_gathers.append(
                pltpu.async_copy(post_sh.at[pidx.at[sl]], prows.at[sl], gsems[j]))

        o0 = sid * b_per_w
        cross = [None] * n_chunks
        final = []
        for j in range(n_chunks):
            tok_gathers[j].wait()
            pos_gathers[j].wait()

            @plsc.parallel_loop(j * _CHUNK, (j + 1) * _CHUNK, step=_UNROLL)
            def add_rows(i):
                for u in range(_UNROLL):
                    for k in range(d // _L):
                        sl = pl.ds(k * _L, _L)
                        plsc.addupdate(trows.at[i + u, sl], prows[i + u, sl])

            sl = pl.ds(j * _CHUNK, _CHUNK)
            cross[j] = pltpu.async_copy(
                trows.at[sl], out_sh.at[pl.ds(o0 + j * _CHUNK, _CHUNK)], csems[j])
            if j > 0:
                cross[j - 1].wait()
                final.append(pltpu.async_copy(
                    out_sh.at[pl.ds(o0 + (j - 1) * _CHUNK, _CHUNK)],
                    out_hbm.at[bb, pl.ds(s0 + (j - 1) * _CHUNK, _CHUNK)],
                    fsem))
        cross[n_chunks - 1].wait()
        final.append(pltpu.async_copy(
            out_sh.at[pl.ds(o0 + (n_chunks - 1) * _CHUNK, _CHUNK)],
            out_hbm.at[bb, pl.ds(s0 + (n_chunks - 1) * _CHUNK, _CHUNK)],
            fsem))
        for f in final:
            f.wait()

    return emb_kernel


def kernel(d_model, max_len, tok_table, pos_table):
    tokens, positions = d_model, max_len
    b, s = tokens.shape
    d = tok_table.shape[1]
    n_chunks = (b * s) // (_NW * _CHUNK)
    fn = _build_lookup(b, s, d, n_chunks, pos_table.shape[0])
    return fn(tokens.astype(jnp.int32), positions.astype(jnp.int32),
              tok_table, pos_table)


# speedup vs baseline: 1.0829x; 1.0829x over previous
import functools

import jax
import jax.numpy as jnp
from jax import lax
from jax.experimental import pallas as pl
from jax.experimental.pallas import tpu as pltpu
from jax.experimental.pallas import tpu_sc as plsc

_INFO = plsc.get_sparse_core_info()
_NC, _NS, _L = _INFO.num_cores, _INFO.num_subcores, _INFO.num_lanes
_NW = _NC * _NS

_CHUNK = 64


def _build_lookup(b, s, d, n_chunks, n_pos):
    b_per_w = n_chunks * _CHUNK
    mesh = plsc.VectorSubcoreMesh(core_axis_name="c", subcore_axis_name="s")

    @functools.partial(
        pl.kernel,
        mesh=mesh,
        out_type=jax.ShapeDtypeStruct((b, s, d), jnp.float32),
        scratch_types=[
            pltpu.VMEM((b_per_w,), jnp.int32),
            pltpu.VMEM((b_per_w,), jnp.int32),
            pltpu.VMEM((b_per_w, d), jnp.float32),
            pltpu.VMEM_SHARED((n_pos, d), jnp.float32),
            pltpu.SemaphoreType.DMA,
            pltpu.SemaphoreType.DMA,
            pltpu.SemaphoreType.DMA,
        ]
        + [pltpu.SemaphoreType.DMA for _ in range(2 * n_chunks)],
    )
    def emb_kernel(tok_hbm, pos_hbm, tokt_hbm, post_hbm, out_hbm,
                   tidx, pidx, trows, post_sh, isem, ssem, osem, *sems):
        tsems, psems = sems[:n_chunks], sems[n_chunks:]
        sid = lax.axis_index("s")
        wid = sid * _NC + lax.axis_index("c")
        bb = wid // (s // b_per_w)
        s0 = (wid % (s // b_per_w)) * b_per_w
        i0 = pltpu.async_copy(tok_hbm.at[bb, pl.ds(s0, b_per_w)], tidx, isem)
        i1 = pltpu.async_copy(pos_hbm.at[bb, pl.ds(s0, b_per_w)], pidx, isem)
        p_per_t = n_pos // _NS
        stg = pltpu.async_copy(
            post_hbm.at[pl.ds(sid * p_per_t, p_per_t)],
            post_sh.at[pl.ds(sid * p_per_t, p_per_t)],
            ssem)
        i0.wait()
        i1.wait()
        tok_gathers = []
        for j in range(n_chunks):
            sl = pl.ds(j * _CHUNK, _CHUNK)
            tok_gathers.append(
                pltpu.async_copy(tokt_hbm.at[tidx.at[sl]], trows.at[sl], tsems[j]))
        stg.wait()
        plsc.subcore_barrier()
        pos_adds = [None] * n_chunks
        stores = []
        for j in range(n_chunks):
            tok_gathers[j].wait()
            sl = pl.ds(j * _CHUNK, _CHUNK)
            pos_adds[j] = pltpu.async_copy(
                post_sh.at[pidx.at[sl]], trows.at[sl], psems[j], add=True)
            if j > 0:
                pos_adds[j - 1].wait()
                sl0 = pl.ds((j - 1) * _CHUNK, _CHUNK)
                stores.append(pltpu.async_copy(
                    trows.at[sl0],
                    out_hbm.at[bb, pl.ds(s0 + (j - 1) * _CHUNK, _CHUNK)],
                    osem))
        pos_adds[n_chunks - 1].wait()
        sl0 = pl.ds((n_chunks - 1) * _CHUNK, _CHUNK)
        stores.append(pltpu.async_copy(
            trows.at[sl0],
            out_hbm.at[bb, pl.ds(s0 + (n_chunks - 1) * _CHUNK, _CHUNK)],
            osem))
        for st in stores:
            st.wait()

    return emb_kernel


def kernel(d_model, max_len, tok_table, pos_table):
    tokens, positions = d_model, max_len
    b, s = tokens.shape
    d = tok_table.shape[1]
    n_chunks = (b * s) // (_NW * _CHUNK)
    fn = _build_lookup(b, s, d, n_chunks, pos_table.shape[0])
    return fn(tokens.astype(jnp.int32), positions.astype(jnp.int32),
              tok_table, pos_table)


# speedup vs baseline: 1.0860x; 1.0029x over previous
import functools

import jax
import jax.numpy as jnp
from jax import lax
from jax.experimental import pallas as pl
from jax.experimental.pallas import tpu as pltpu
from jax.experimental.pallas import tpu_sc as plsc

_INFO = plsc.get_sparse_core_info()
_NC, _NS, _L = _INFO.num_cores, _INFO.num_subcores, _INFO.num_lanes
_NW = _NC * _NS

_CHUNK = 32


def _build_lookup(b, s, d, n_chunks, n_pos):
    b_per_w = n_chunks * _CHUNK
    mesh = plsc.VectorSubcoreMesh(core_axis_name="c", subcore_axis_name="s")

    @functools.partial(
        pl.kernel,
        mesh=mesh,
        out_type=jax.ShapeDtypeStruct((b, s, d), jnp.float32),
        scratch_types=[
            pltpu.VMEM((b_per_w,), jnp.int32),
            pltpu.VMEM((b_per_w,), jnp.int32),
            pltpu.VMEM((b_per_w, d), jnp.float32),
            pltpu.VMEM_SHARED((n_pos, d), jnp.float32),
            pltpu.SemaphoreType.DMA,
            pltpu.SemaphoreType.DMA,
            pltpu.SemaphoreType.DMA,
        ]
        + [pltpu.SemaphoreType.DMA for _ in range(2 * n_chunks)],
    )
    def emb_kernel(tok_hbm, pos_hbm, tokt_hbm, post_hbm, out_hbm,
                   tidx, pidx, trows, post_sh, isem, ssem, osem, *sems):
        tsems, psems = sems[:n_chunks], sems[n_chunks:]
        sid = lax.axis_index("s")
        wid = sid * _NC + lax.axis_index("c")
        bb = wid // (s // b_per_w)
        s0 = (wid % (s // b_per_w)) * b_per_w
        i0 = pltpu.async_copy(tok_hbm.at[bb, pl.ds(s0, b_per_w)], tidx, isem)
        i1 = pltpu.async_copy(pos_hbm.at[bb, pl.ds(s0, b_per_w)], pidx, isem)
        p_per_t = n_pos // _NS
        stg = pltpu.async_copy(
            post_hbm.at[pl.ds(sid * p_per_t, p_per_t)],
            post_sh.at[pl.ds(sid * p_per_t, p_per_t)],
            ssem)
        i0.wait()
        i1.wait()
        tok_gathers = []
        for j in range(n_chunks):
            sl = pl.ds(j * _CHUNK, _CHUNK)
            tok_gathers.append(
                pltpu.async_copy(tokt_hbm.at[tidx.at[sl]], trows.at[sl], tsems[j]))
        stg.wait()
        plsc.subcore_barrier()
        pos_adds = [None] * n_chunks
        stores = []
        for j in range(n_chunks):
            tok_gathers[j].wait()
            sl = pl.ds(j * _CHUNK, _CHUNK)
            pos_adds[j] = pltpu.async_copy(
                post_sh.at[pidx.at[sl]], trows.at[sl], psems[j], add=True)
            if j > 0:
                pos_adds[j - 1].wait()
                sl0 = pl.ds((j - 1) * _CHUNK, _CHUNK)
                stores.append(pltpu.async_copy(
                    trows.at[sl0],
                    out_hbm.at[bb, pl.ds(s0 + (j - 1) * _CHUNK, _CHUNK)],
                    osem))
        pos_adds[n_chunks - 1].wait()
        sl0 = pl.ds((n_chunks - 1) * _CHUNK, _CHUNK)
        stores.append(pltpu.async_copy(
            trows.at[sl0],
            out_hbm.at[bb, pl.ds(s0 + (n_chunks - 1) * _CHUNK, _CHUNK)],
            osem))
        for st in stores:
            st.wait()

    return emb_kernel


def kernel(d_model, max_len, tok_table, pos_table):
    tokens, positions = d_model, max_len
    b, s = tokens.shape
    d = tok_table.shape[1]
    n_chunks = (b * s) // (_NW * _CHUNK)
    fn = _build_lookup(b, s, d, n_chunks, pos_table.shape[0])
    return fn(tokens.astype(jnp.int32), positions.astype(jnp.int32),
              tok_table, pos_table)


# speedup vs baseline: 1.0895x; 1.0032x over previous
import functools

import jax
import jax.numpy as jnp
from jax import lax
from jax.experimental import pallas as pl
from jax.experimental.pallas import tpu as pltpu
from jax.experimental.pallas import tpu_sc as plsc

_INFO = plsc.get_sparse_core_info()
_NC, _NS, _L = _INFO.num_cores, _INFO.num_subcores, _INFO.num_lanes
_NW = _NC * _NS

_CHUNK = 32


def _build_lookup(b, s, d, n_chunks, n_pos):
    b_per_w = n_chunks * _CHUNK
    mesh = plsc.VectorSubcoreMesh(core_axis_name="c", subcore_axis_name="s")

    @functools.partial(
        pl.kernel,
        mesh=mesh,
        out_type=jax.ShapeDtypeStruct((b, s, d), jnp.float32),
        scratch_types=[
            pltpu.VMEM((b_per_w,), jnp.int32),
            pltpu.VMEM((b_per_w,), jnp.int32),
            pltpu.VMEM((b_per_w, d), jnp.float32),
            pltpu.VMEM_SHARED((n_pos, d), jnp.float32),
            pltpu.SemaphoreType.DMA,
            pltpu.SemaphoreType.DMA,
            pltpu.SemaphoreType.DMA,
        ]
        + [pltpu.SemaphoreType.DMA for _ in range(2 * n_chunks)],
    )
    def emb_kernel(tok_hbm, pos_hbm, tokt_hbm, post_hbm, out_hbm,
                   tidx, pidx, trows, post_sh, isem, ssem, osem, *sems):
        tsems, psems = sems[:n_chunks], sems[n_chunks:]
        sid = lax.axis_index("s")
        wid = sid * _NC + lax.axis_index("c")
        bb = wid // (s // b_per_w)
        s0 = (wid % (s // b_per_w)) * b_per_w
        i0 = pltpu.async_copy(tok_hbm.at[bb, pl.ds(s0, b_per_w)], tidx, isem)
        i1 = pltpu.async_copy(pos_hbm.at[bb, pl.ds(s0, b_per_w)], pidx, isem)
        p_per_t = n_pos // _NS
        stg = pltpu.async_copy(
            post_hbm.at[pl.ds(sid * p_per_t, p_per_t)],
            post_sh.at[pl.ds(sid * p_per_t, p_per_t)],
            ssem)
        i0.wait()
        tok_gathers = []
        for j in range(n_chunks):
            sl = pl.ds(j * _CHUNK, _CHUNK)
            tok_gathers.append(
                pltpu.async_copy(tokt_hbm.at[tidx.at[sl]], trows.at[sl], tsems[j]))
        i1.wait()
        stg.wait()
        plsc.subcore_barrier()
        pos_adds = [None] * n_chunks
        stores = []
        for j in range(n_chunks):
            tok_gathers[j].wait()
            sl = pl.ds(j * _CHUNK, _CHUNK)
            pos_adds[j] = pltpu.async_copy(
                post_sh.at[pidx.at[sl]], trows.at[sl], psems[j], add=True)
            if j > 0:
                pos_adds[j - 1].wait()
                sl0 = pl.ds((j - 1) * _CHUNK, _CHUNK)
                stores.append(pltpu.async_copy(
                    trows.at[sl0],
                    out_hbm.at[bb, pl.ds(s0 + (j - 1) * _CHUNK, _CHUNK)],
                    osem))
        pos_adds[n_chunks - 1].wait()
        sl0 = pl.ds((n_chunks - 1) * _CHUNK, _CHUNK)
        stores.append(pltpu.async_copy(
            trows.at[sl0],
            out_hbm.at[bb, pl.ds(s0 + (n_chunks - 1) * _CHUNK, _CHUNK)],
            osem))
        for st in stores:
            st.wait()

    return emb_kernel


def kernel(d_model, max_len, tok_table, pos_table):
    tokens, positions = d_model, max_len
    b, s = tokens.shape
    d = tok_table.shape[1]
    n_chunks = (b * s) // (_NW * _CHUNK)
    fn = _build_lookup(b, s, d, n_chunks, pos_table.shape[0])
    return fn(tokens.astype(jnp.int32), positions.astype(jnp.int32),
              tok_table, pos_table)
